# baseline (speedup 1.0000x reference)
import jax
import jax.numpy as jnp
from jax.experimental import pallas as pl


_BLOCK_ROWS = 128


def _mask_kernel(w_ref, t_ref, o_ref):
    w = w_ref[...]
    s = jax.nn.sigmoid(t_ref[...])
    o_ref[...] = jnp.sign(w) * jnp.maximum(jnp.abs(w) - s, 0.0)


def kernel(weight, threshold, alpha):
    rows, cols = weight.shape
    grid = (rows // _BLOCK_ROWS,)
    return pl.pallas_call(
        _mask_kernel,
        grid=grid,
        in_specs=[
            pl.BlockSpec((_BLOCK_ROWS, cols), lambda i: (i, 0)),
            pl.BlockSpec((_BLOCK_ROWS, 1), lambda i: (i, 0)),
        ],
        out_specs=pl.BlockSpec((_BLOCK_ROWS, cols), lambda i: (i, 0)),
        out_shape=jax.ShapeDtypeStruct((rows, cols), weight.dtype),
    )(weight, threshold)


# speedup vs baseline: 1.2014x; 1.2014x over previous
import jax
import jax.numpy as jnp
from jax.experimental import pallas as pl


_BLOCK_ROWS = 128


def _mask_kernel(w_ref, t_ref, o_ref):
    w = w_ref[...]
    s = jax.nn.sigmoid(t_ref[...])
    o_ref[...] = jnp.maximum(w - s, 0.0) + jnp.minimum(w + s, 0.0)


def kernel(weight, threshold, alpha):
    rows, cols = weight.shape
    grid = (rows // _BLOCK_ROWS,)
    return pl.pallas_call(
        _mask_kernel,
        grid=grid,
        in_specs=[
            pl.BlockSpec((_BLOCK_ROWS, cols), lambda i: (i, 0)),
            pl.BlockSpec((_BLOCK_ROWS, 1), lambda i: (i, 0)),
        ],
        out_specs=pl.BlockSpec((_BLOCK_ROWS, cols), lambda i: (i, 0)),
        out_shape=jax.ShapeDtypeStruct((rows, cols), weight.dtype),
    )(weight, threshold)
